# baseline (speedup 1.0000x reference)
import functools

import numpy as np
import jax
import jax.numpy as jnp
from jax import lax
from jax.experimental import pallas as pl
from jax.experimental.pallas import tpu as pltpu
from jax.experimental.pallas import tpu_sc as plsc

_B = 1024
_L = 200
_D = 128
_NC = 2
_NS = 16
_NW = _NC * _NS
_BW = _B // _NW
_NH = 2
_HB = _BW // _NH
_KO = 8
_NO = _L // _KO
_STEPS = _NO * _NH
_RPS = _HB * _KO
_ROWS_PER_W = _BW * _L
_LANES = 16
_NBUF = 3


def _sin_pe(seq_len, d_model):
    pos = np.arange(seq_len, dtype=np.float32)[:, None]
    div = np.exp(np.arange(0, d_model, 2, dtype=np.float32)
                 * (-np.log(10000.0) / d_model))
    ang = pos * div[None, :]
    pe = np.zeros((seq_len, d_model), dtype=np.float32)
    pe[:, 0::2] = np.sin(ang)
    pe[:, 1::2] = np.cos(ang)
    return pe


_PE = _sin_pe(_L, _D)


def _make_sc_kernel():
    mesh = plsc.VectorSubcoreMesh(core_axis_name="c", subcore_axis_name="s")

    @functools.partial(
        pl.kernel,
        mesh=mesh,
        out_type=(
            jax.ShapeDtypeStruct((_B, _L, _D), jnp.float32),
            jax.ShapeDtypeStruct((_B * _L,), jnp.int32),
        ),
        scratch_types=[
            pltpu.VMEM((_STEPS, _RPS), jnp.int32),
            pltpu.VMEM((_NBUF, _RPS, _D), jnp.float32),
            pltpu.VMEM((_L, _D), jnp.float32),
            pltpu.VMEM((_ROWS_PER_W,), jnp.int32),
            pltpu.SemaphoreType.DMA((_NBUF,)),
            pltpu.SemaphoreType.DMA((_NBUF,)),
        ],
    )
    def emb_kernel(xp_hbm, table_hbm, pe_hbm,
                   out_hbm, mask_hbm,
                   idx_v, rows_v, pe_v, mask_v, sg, ss):
        wid = lax.axis_index("s") * _NC + lax.axis_index("c")
        base = wid * _ROWS_PER_W
        wb0 = wid * _BW

        pltpu.sync_copy(xp_hbm.at[wid], idx_v)
        pltpu.sync_copy(pe_hbm, pe_v)

        def start_gather(t, bu):
            pltpu.async_copy(
                table_hbm.at[idx_v.at[t]], rows_v.at[bu], sg.at[bu])

        def wait_gather(bu):
            pltpu.make_async_copy(
                table_hbm.at[pl.ds(0, _RPS)], rows_v.at[bu], sg.at[bu]).wait()

        def start_scatter(t, bu):
            o = t // _NH
            h = t % _NH
            lk = o * _KO
            bb0 = wb0 + h * _HB
            for bi in range(_HB):
                pltpu.async_copy(
                    rows_v.at[bu, pl.ds(bi * _KO, _KO)],
                    out_hbm.at[bb0 + bi, pl.ds(lk, _KO)], ss.at[bu])

        def wait_scatter(bu):
            pltpu.make_async_copy(
                rows_v.at[bu], out_hbm.at[0, pl.ds(0, _RPS)], ss.at[bu]).wait()

        start_gather(0, 0)

        def step_body(t, carry):
            bu = t % _NBUF
            bn = (t + 1) % _NBUF
            o = t // _NH
            lk = o * _KO

            @pl.when(t >= _NBUF - 1)
            def _():
                wait_scatter(bn)

            @pl.when(t + 1 < _STEPS)
            def _():
                start_gather(t + 1, bn)

            wait_gather(bu)

            for lj in range(_KO):
                pe_regs = [pe_v[lk + lj, pl.ds(c * _LANES, _LANES)]
                           for c in range(_D // _LANES)]

                @plsc.parallel_loop(0, _HB, unroll=4)
                def add_body(bi):
                    r = bi * _KO + lj
                    for c in range(_D // _LANES):
                        plsc.addupdate(
                            rows_v.at[bu, r, pl.ds(c * _LANES, _LANES)],
                            pe_regs[c])

            start_scatter(t, bu)

            for k in range(_RPS // _LANES):
                v = idx_v[t, pl.ds(k * _LANES, _LANES)]
                mask_v[pl.ds(t * _RPS + k * _LANES, _LANES)] = jnp.where(
                    v == 0, jnp.full((_LANES,), 1, jnp.int32),
                    jnp.full((_LANES,), 0, jnp.int32))
            return carry

        lax.fori_loop(0, _STEPS, step_body, 0)

        pltpu.sync_copy(mask_v, mask_hbm.at[pl.ds(base, _ROWS_PER_W)])

        wait_scatter((_STEPS - 2) % _NBUF)
        wait_scatter((_STEPS - 1) % _NBUF)

    return emb_kernel


_EMB_KERNEL = _make_sc_kernel()


def kernel(x, emb_table):
    x32 = x.astype(jnp.int32)
    xp = x32.reshape(_NW, _NH, _HB, _NO, _KO)
    xp = xp.transpose(0, 3, 1, 2, 4).reshape(_NW, _STEPS, _RPS)
    pe = jnp.asarray(_PE)
    h, mask_perm = _EMB_KERNEL(xp, emb_table, pe)
    m = mask_perm.reshape(_NW, _NO, _NH, _HB, _KO)
    m = m.transpose(0, 2, 3, 1, 4).reshape(_B, _L)
    padding_mask = m.astype(bool)
    return h, padding_mask


# speedup vs baseline: 1.1362x; 1.1362x over previous
import functools

import numpy as np
import jax
import jax.numpy as jnp
from jax import lax
from jax.experimental import pallas as pl
from jax.experimental.pallas import tpu as pltpu
from jax.experimental.pallas import tpu_sc as plsc

_B = 1024
_L = 200
_D = 128
_NC = 2
_NS = 16
_NW = _NC * _NS
_BW = _B // _NW
_HB = 4
_NH = _BW // _HB
_KO = 40
_NO = _L // _KO
_STEPS = _NO * _NH
_RPS = _HB * _KO
_ROWS_PER_W = _BW * _L
_LANES = 16
_NBUF = 4


def _sin_pe(seq_len, d_model):
    pos = np.arange(seq_len, dtype=np.float32)[:, None]
    div = np.exp(np.arange(0, d_model, 2, dtype=np.float32)
                 * (-np.log(10000.0) / d_model))
    ang = pos * div[None, :]
    pe = np.zeros((seq_len, d_model), dtype=np.float32)
    pe[:, 0::2] = np.sin(ang)
    pe[:, 1::2] = np.cos(ang)
    return pe


_PE = _sin_pe(_L, _D)


def _make_sc_kernel():
    mesh = plsc.VectorSubcoreMesh(core_axis_name="c", subcore_axis_name="s")

    @functools.partial(
        pl.kernel,
        mesh=mesh,
        out_type=(
            jax.ShapeDtypeStruct((_B, _L, _D), jnp.float32),
            jax.ShapeDtypeStruct((_B * _L,), jnp.int32),
        ),
        scratch_types=[
            pltpu.VMEM((_STEPS, _RPS), jnp.int32),
            pltpu.VMEM((_NBUF, _RPS, _D), jnp.float32),
            pltpu.VMEM((_L, _D), jnp.float32),
            pltpu.VMEM((_ROWS_PER_W,), jnp.int32),
            pltpu.SemaphoreType.DMA((_NBUF,)),
            pltpu.SemaphoreType.DMA((_NBUF,)),
        ],
    )
    def emb_kernel(xp_hbm, table_hbm, pe_hbm,
                   out_hbm, mask_hbm,
                   idx_v, rows_v, pe_v, mask_v, sg, ss):
        wid = lax.axis_index("s") * _NC + lax.axis_index("c")
        base = wid * _ROWS_PER_W
        wb0 = wid * _BW

        pltpu.sync_copy(xp_hbm.at[wid], idx_v)
        pltpu.sync_copy(pe_hbm, pe_v)

        def start_gather(t, bu):
            pltpu.async_copy(
                table_hbm.at[idx_v.at[t, pl.ds(0, 128)]],
                rows_v.at[bu, pl.ds(0, 128)], sg.at[bu])
            pltpu.async_copy(
                table_hbm.at[idx_v.at[t, pl.ds(128, _RPS - 128)]],
                rows_v.at[bu, pl.ds(128, _RPS - 128)], sg.at[bu])

        def wait_gather(bu):
            pltpu.make_async_copy(
                table_hbm.at[pl.ds(0, _RPS)],
                rows_v.at[bu], sg.at[bu]).wait()

        def start_scatter(t, bu):
            o = t // _NH
            h = t % _NH
            lk = o * _KO
            bb0 = wb0 + h * _HB
            for bi in range(_HB):
                pltpu.async_copy(
                    rows_v.at[bu, pl.ds(bi * _KO, _KO)],
                    out_hbm.at[bb0 + bi, pl.ds(lk, _KO)], ss.at[bu])

        def wait_scatter(bu):
            pltpu.make_async_copy(
                rows_v.at[bu], out_hbm.at[0, pl.ds(0, _RPS)], ss.at[bu]).wait()

        start_gather(0, 0)

        def step_body(t, carry):
            bu = t % _NBUF
            bn = (t + 1) % _NBUF
            o = t // _NH
            lk = o * _KO

            @pl.when(t >= _NBUF - 1)
            def _():
                wait_scatter(bn)

            @pl.when(t + 1 < _STEPS)
            def _():
                start_gather(t + 1, bn)

            wait_gather(bu)

            for lj in range(_KO):
                pe_regs = [pe_v[lk + lj, pl.ds(c * _LANES, _LANES)]
                           for c in range(_D // _LANES)]

                @plsc.parallel_loop(0, _HB, unroll=_HB)
                def add_body(bi):
                    r = bi * _KO + lj
                    for c in range(_D // _LANES):
                        plsc.addupdate(
                            rows_v.at[bu, r, pl.ds(c * _LANES, _LANES)],
                            pe_regs[c])

            start_scatter(t, bu)

            for k in range(_RPS // _LANES):
                v = idx_v[t, pl.ds(k * _LANES, _LANES)]
                mask_v[pl.ds(t * _RPS + k * _LANES, _LANES)] = jnp.where(
                    v == 0, jnp.full((_LANES,), 1, jnp.int32),
                    jnp.full((_LANES,), 0, jnp.int32))
            return carry

        lax.fori_loop(0, _STEPS, step_body, 0)

        pltpu.sync_copy(mask_v, mask_hbm.at[pl.ds(base, _ROWS_PER_W)])

        for d in range(1, _NBUF):
            wait_scatter((_STEPS - d) % _NBUF)

    return emb_kernel


_EMB_KERNEL = _make_sc_kernel()


def kernel(x, emb_table):
    x32 = x.astype(jnp.int32)
    xp = x32.reshape(_NW, _NH, _HB, _NO, _KO)
    xp = xp.transpose(0, 3, 1, 2, 4).reshape(_NW, _STEPS, _RPS)
    pe = jnp.asarray(_PE)
    h, mask_perm = _EMB_KERNEL(xp, emb_table, pe)
    m = mask_perm.reshape(_NW, _NO, _NH, _HB, _KO)
    m = m.transpose(0, 2, 3, 1, 4).reshape(_B, _L)
    padding_mask = m.astype(bool)
    return h, padding_mask


# speedup vs baseline: 1.2260x; 1.0790x over previous
import functools

import numpy as np
import jax
import jax.numpy as jnp
from jax import lax
from jax.experimental import pallas as pl
from jax.experimental.pallas import tpu as pltpu
from jax.experimental.pallas import tpu_sc as plsc

_B = 1024
_L = 200
_D = 128
_NC = 2
_NS = 16
_NW = _NC * _NS
_BW = _B // _NW
_HB = 4
_NH = _BW // _HB
_KO = 40
_NO = _L // _KO
_STEPS = _NO * _NH
_RPS = _HB * _KO
_ROWS_PER_W = _BW * _L
_LANES = 16
_NBUF = 4


def _sin_pe(seq_len, d_model):
    pos = np.arange(seq_len, dtype=np.float32)[:, None]
    div = np.exp(np.arange(0, d_model, 2, dtype=np.float32)
                 * (-np.log(10000.0) / d_model))
    ang = pos * div[None, :]
    pe = np.zeros((seq_len, d_model), dtype=np.float32)
    pe[:, 0::2] = np.sin(ang)
    pe[:, 1::2] = np.cos(ang)
    return pe


_PE = _sin_pe(_L, _D)


def _make_sc_kernel():
    mesh = plsc.VectorSubcoreMesh(core_axis_name="c", subcore_axis_name="s")

    @functools.partial(
        pl.kernel,
        mesh=mesh,
        out_type=(
            jax.ShapeDtypeStruct((_B, _L, _D), jnp.float32),
            jax.ShapeDtypeStruct((_B * _L,), jnp.int32),
        ),
        scratch_types=[
            pltpu.VMEM((_ROWS_PER_W,), jnp.int32),
            pltpu.VMEM((_NBUF, _RPS, _D), jnp.float32),
            pltpu.VMEM((_L, _D), jnp.float32),
            pltpu.VMEM((_ROWS_PER_W,), jnp.int32),
            pltpu.SemaphoreType.DMA((_NBUF,)),
            pltpu.SemaphoreType.DMA((_NBUF,)),
        ],
    )
    def emb_kernel(x_hbm, table_hbm, pe_hbm,
                   out_hbm, mask_hbm,
                   idx_v, rows_v, pe_v, mask_v, sg, ss):
        wid = lax.axis_index("s") * _NC + lax.axis_index("c")
        base = wid * _ROWS_PER_W
        wb0 = wid * _BW

        pltpu.sync_copy(x_hbm.at[pl.ds(base, _ROWS_PER_W)], idx_v)
        pltpu.sync_copy(pe_hbm, pe_v)

        def start_gather(t, bu):
            o = t // _NH
            h = t % _NH
            lk = o * _KO
            for bi in range(_HB):
                pltpu.async_copy(
                    table_hbm.at[idx_v.at[pl.ds((h * _HB + bi) * _L + lk, _KO)]],
                    rows_v.at[bu, pl.ds(bi * _KO, _KO)], sg.at[bu])

        def wait_gather(bu):
            pltpu.make_async_copy(
                table_hbm.at[pl.ds(0, _RPS)],
                rows_v.at[bu], sg.at[bu]).wait()

        def start_scatter(t, bu):
            o = t // _NH
            h = t % _NH
            lk = o * _KO
            bb0 = wb0 + h * _HB
            for bi in range(_HB):
                pltpu.async_copy(
                    rows_v.at[bu, pl.ds(bi * _KO, _KO)],
                    out_hbm.at[bb0 + bi, pl.ds(lk, _KO)], ss.at[bu])

        def wait_scatter(bu):
            pltpu.make_async_copy(
                rows_v.at[bu], out_hbm.at[0, pl.ds(0, _RPS)], ss.at[bu]).wait()

        start_gather(0, 0)

        def step_body(t, carry):
            bu = t % _NBUF
            bn = (t + 1) % _NBUF
            o = t // _NH
            lk = o * _KO

            @pl.when(t >= _NBUF - 1)
            def _():
                wait_scatter(bn)

            @pl.when(t + 1 < _STEPS)
            def _():
                start_gather(t + 1, bn)

            wait_gather(bu)

            for lj in range(_KO):
                pe_regs = [pe_v[lk + lj, pl.ds(c * _LANES, _LANES)]
                           for c in range(_D // _LANES)]

                @plsc.parallel_loop(0, _HB, unroll=_HB)
                def add_body(bi):
                    r = bi * _KO + lj
                    for c in range(_D // _LANES):
                        plsc.addupdate(
                            rows_v.at[bu, r, pl.ds(c * _LANES, _LANES)],
                            pe_regs[c])

            start_scatter(t, bu)
            return carry

        lax.fori_loop(0, _STEPS, step_body, 0)

        def mask_body(i, carry):
            v = idx_v[pl.ds(i * _LANES, _LANES)]
            mask_v[pl.ds(i * _LANES, _LANES)] = jnp.where(
                v == 0, jnp.full((_LANES,), 1, jnp.int32),
                jnp.full((_LANES,), 0, jnp.int32))
            return carry

        lax.fori_loop(0, _ROWS_PER_W // _LANES, mask_body, 0)
        pltpu.sync_copy(mask_v, mask_hbm.at[pl.ds(base, _ROWS_PER_W)])

        for d in range(1, _NBUF):
            wait_scatter((_STEPS - d) % _NBUF)

    return emb_kernel


_EMB_KERNEL = _make_sc_kernel()


def kernel(x, emb_table):
    x32 = x.astype(jnp.int32).reshape(_B * _L)
    pe = jnp.asarray(_PE)
    h, mask_i32 = _EMB_KERNEL(x32, emb_table, pe)
    padding_mask = mask_i32.reshape(_B, _L).astype(bool)
    return h, padding_mask


# speedup vs baseline: 1.3088x; 1.0676x over previous
import functools

import numpy as np
import jax
import jax.numpy as jnp
from jax import lax
from jax.experimental import pallas as pl
from jax.experimental.pallas import tpu as pltpu
from jax.experimental.pallas import tpu_sc as plsc

_B = 1024
_L = 200
_D = 128
_NC = 2
_NS = 16
_NW = _NC * _NS
_BW = _B // _NW
_HB = 4
_NH = _BW // _HB
_KO = 40
_NO = _L // _KO
_STEPS = _NO * _NH
_RPS = _HB * _KO
_ROWS_PER_W = _BW * _L
_LANES = 16
_NBUF = 4


def _sin_pe(seq_len, d_model):
    pos = np.arange(seq_len, dtype=np.float32)[:, None]
    div = np.exp(np.arange(0, d_model, 2, dtype=np.float32)
                 * (-np.log(10000.0) / d_model))
    ang = pos * div[None, :]
    pe = np.zeros((seq_len, d_model), dtype=np.float32)
    pe[:, 0::2] = np.sin(ang)
    pe[:, 1::2] = np.cos(ang)
    return pe


_PE = _sin_pe(_L, _D)


def _make_sc_kernel():
    mesh = plsc.VectorSubcoreMesh(core_axis_name="c", subcore_axis_name="s")

    @functools.partial(
        pl.kernel,
        mesh=mesh,
        out_type=(
            jax.ShapeDtypeStruct((_B, _L, _D), jnp.float32),
            jax.ShapeDtypeStruct((_B * _L,), jnp.int32),
        ),
        scratch_types=[
            pltpu.VMEM((_ROWS_PER_W,), jnp.int32),
            pltpu.VMEM((_NBUF, _RPS, _D), jnp.float32),
            pltpu.VMEM((_L, _D), jnp.float32),
            pltpu.VMEM((_ROWS_PER_W,), jnp.int32),
            pltpu.SemaphoreType.DMA((_NBUF,)),
            pltpu.SemaphoreType.DMA((_NBUF,)),
        ],
    )
    def emb_kernel(x_hbm, table_hbm, pe_hbm,
                   out_hbm, mask_hbm,
                   idx_v, rows_v, pe_v, mask_v, sg, ss):
        wid = lax.axis_index("s") * _NC + lax.axis_index("c")
        base = wid * _ROWS_PER_W
        wb0 = wid * _BW

        pltpu.sync_copy(x_hbm.at[pl.ds(base, _ROWS_PER_W)], idx_v)
        pltpu.sync_copy(pe_hbm, pe_v)

        def start_gather(t, bu):
            o = t // _NH
            h = t % _NH
            lk = o * _KO
            for bi in range(_HB):
                pltpu.async_copy(
                    table_hbm.at[idx_v.at[pl.ds((h * _HB + bi) * _L + lk, _KO)]],
                    rows_v.at[bu, pl.ds(bi * _KO, _KO)], sg.at[bu])

        def wait_gather(bu):
            pltpu.make_async_copy(
                table_hbm.at[pl.ds(0, _RPS)],
                rows_v.at[bu], sg.at[bu]).wait()

        def start_scatter(t, bu):
            o = t // _NH
            h = t % _NH
            lk = o * _KO
            bb0 = wb0 + h * _HB
            for bi in range(_HB):
                pltpu.async_copy(
                    rows_v.at[bu, pl.ds(bi * _KO, _KO)],
                    out_hbm.at[bb0 + bi, pl.ds(lk, _KO)], ss.at[bu])

        def wait_scatter(bu):
            pltpu.make_async_copy(
                rows_v.at[bu], out_hbm.at[0, pl.ds(0, _RPS)], ss.at[bu]).wait()

        start_gather(0, 0)
        start_gather(1, 1)

        def step_body(t, carry):
            bu = t % _NBUF
            bn = (t + 2) % _NBUF
            o = t // _NH
            lk = o * _KO

            @pl.when(t >= _NBUF - 2)
            def _():
                wait_scatter(bn)

            @pl.when(t + 2 < _STEPS)
            def _():
                start_gather(t + 2, bn)

            wait_gather(bu)

            for lj in range(_KO):
                pe_regs = [pe_v[lk + lj, pl.ds(c * _LANES, _LANES)]
                           for c in range(_D // _LANES)]

                @plsc.parallel_loop(0, _HB, unroll=_HB)
                def add_body(bi):
                    r = bi * _KO + lj
                    for c in range(_D // _LANES):
                        plsc.addupdate(
                            rows_v.at[bu, r, pl.ds(c * _LANES, _LANES)],
                            pe_regs[c])

            start_scatter(t, bu)
            return carry

        lax.fori_loop(0, _STEPS, step_body, 0)

        def mask_body(i, carry):
            v = idx_v[pl.ds(i * _LANES, _LANES)]
            mask_v[pl.ds(i * _LANES, _LANES)] = jnp.where(
                v == 0, jnp.full((_LANES,), 1, jnp.int32),
                jnp.full((_LANES,), 0, jnp.int32))
            return carry

        lax.fori_loop(0, _ROWS_PER_W // _LANES, mask_body, 0)
        pltpu.sync_copy(mask_v, mask_hbm.at[pl.ds(base, _ROWS_PER_W)])

        for d in range(1, 3):
            wait_scatter((_STEPS - d) % _NBUF)

    return emb_kernel


_EMB_KERNEL = _make_sc_kernel()


def kernel(x, emb_table):
    x32 = x.astype(jnp.int32).reshape(_B * _L)
    pe = jnp.asarray(_PE)
    h, mask_i32 = _EMB_KERNEL(x32, emb_table, pe)
    padding_mask = mask_i32.reshape(_B, _L).astype(bool)
    return h, padding_mask
